# baseline (speedup 1.0000x reference)
import functools

import jax
import jax.numpy as jnp
from jax.experimental import pallas as pl
from jax.experimental.pallas import tpu as pltpu

N_NODES = 100000
N_EDGES = 1600000
F = 32

_NODE_BLK = 2000
_EDGE_BLK = 8000


def _node_tables_body(h_ref, pos_ref, wh_ref, wp_ref, b_ref, a_ref, b_out_ref):
    pq = jax.lax.dot_general(pos_ref[...], wp_ref[...], (((1,), (0,)), ((), ())),
                             preferred_element_type=jnp.float32)
    hq = jax.lax.dot_general(h_ref[...], wh_ref[...], (((1,), (0,)), ((), ())),
                             preferred_element_type=jnp.float32)
    a_ref[...] = hq + pq + b_ref[...]
    b_out_ref[...] = pq


def _node_tables(h, pos, Wh, Wp, b):
    n = h.shape[0]
    k = h.shape[1]
    grid = n // _NODE_BLK
    return pl.pallas_call(
        _node_tables_body,
        grid=(grid,),
        in_specs=[
            pl.BlockSpec((_NODE_BLK, k), lambda i: (i, 0)),
            pl.BlockSpec((_NODE_BLK, 3), lambda i: (i, 0)),
            pl.BlockSpec((k, F), lambda i: (0, 0)),
            pl.BlockSpec((3, F), lambda i: (0, 0)),
            pl.BlockSpec((1, F), lambda i: (0, 0)),
        ],
        out_specs=[
            pl.BlockSpec((_NODE_BLK, F), lambda i: (i, 0)),
            pl.BlockSpec((_NODE_BLK, F), lambda i: (i, 0)),
        ],
        out_shape=[
            jax.ShapeDtypeStruct((n, F), jnp.float32),
            jax.ShapeDtypeStruct((n, F), jnp.float32),
        ],
    )(h, pos, Wh, Wp, b[None, :])


def _edge_mlp_body(asrc_ref, bdst_ref, wb_ref, bb_ref, out_ref):
    e = jnp.maximum(asrc_ref[...] - bdst_ref[...], 0.0)
    out_ref[...] = jax.lax.dot_general(
        e, wb_ref[...], (((1,), (0,)), ((), ())),
        preferred_element_type=jnp.float32) + bb_ref[...]


def _edge_mlp(asrc, bdst, Wb, bb):
    e = asrc.shape[0]
    grid = e // _EDGE_BLK
    return pl.pallas_call(
        _edge_mlp_body,
        grid=(grid,),
        in_specs=[
            pl.BlockSpec((_EDGE_BLK, F), lambda i: (i, 0)),
            pl.BlockSpec((_EDGE_BLK, F), lambda i: (i, 0)),
            pl.BlockSpec((F, F), lambda i: (0, 0)),
            pl.BlockSpec((1, F), lambda i: (0, 0)),
        ],
        out_specs=pl.BlockSpec((_EDGE_BLK, F), lambda i: (i, 0)),
        out_shape=jax.ShapeDtypeStruct((e, F), jnp.float32),
    )(asrc, bdst, Wb, bb[None, :])


def _layer(h, pos, src, dst, Wh, Wp, ba, Wb, bb):
    A, B = _node_tables(h, pos, Wh, Wp, ba)
    asrc = jnp.take(A, src, axis=0)
    bdst = jnp.take(B, dst, axis=0)
    msg = _edge_mlp(asrc, bdst, Wb, bb)
    agg = jax.ops.segment_max(msg, dst, num_segments=N_NODES)
    return jnp.maximum(agg, 0.0)


def _pool_classify_body(h_ref, wc_ref, bc_ref, out_ref):
    out_ref[...] = jax.lax.dot_general(
        h_ref[...], wc_ref[...], (((1,), (0,)), ((), ())),
        preferred_element_type=jnp.float32) + bc_ref[...]


def kernel(pos, edge_index, batch, W1a, b1a, W1b, b1b, W2a, b2a, W2b, b2b, Wc, bc):
    src = edge_index[0]
    dst = edge_index[1]
    h = _layer(pos, pos, src, dst, W1a[:3], W1a[3:], b1a, W1b, b1b)
    h = _layer(h, pos, src, dst, W2a[:F], W2a[F:], b2a, W2b, b2b)
    pooled = jnp.maximum(jax.ops.segment_max(h, batch, num_segments=64), 0.0)
    nc = Wc.shape[1]
    out = pl.pallas_call(
        _pool_classify_body,
        in_specs=[
            pl.BlockSpec((64, F), lambda: (0, 0)),
            pl.BlockSpec((F, nc), lambda: (0, 0)),
            pl.BlockSpec((1, nc), lambda: (0, 0)),
        ],
        out_specs=pl.BlockSpec((64, nc), lambda: (0, 0)),
        out_shape=jax.ShapeDtypeStruct((64, nc), jnp.float32),
    )(pooled, Wc, bc[None, :])
    return out


# speedup vs baseline: 2.3384x; 2.3384x over previous
import functools

import jax
import jax.numpy as jnp
from jax import lax
from jax.experimental import pallas as pl
from jax.experimental.pallas import tpu as pltpu
from jax.experimental.pallas import tpu_sc as plsc

N_NODES = 100000
N_EDGES = 1600000
F = 32

_NODE_BLK = 2000
_EDGE_BLK = 8000

_NC = 2
_NS = 16
_NW = _NC * _NS
_EPT = N_EDGES // _NW
_GCH = 1000
_GNCH = _EPT // _GCH


def _gather_diff_relu(A, B, src, dst):
    mesh = plsc.VectorSubcoreMesh(core_axis_name="c", subcore_axis_name="s")

    @functools.partial(
        pl.kernel, mesh=mesh,
        out_type=jax.ShapeDtypeStruct((N_EDGES, F), jnp.float32),
        compiler_params=pltpu.CompilerParams(use_tc_tiling_on_sc=False),
        scratch_types=[
            pltpu.VMEM((_GCH,), jnp.int32),
            pltpu.VMEM((_GCH,), jnp.int32),
            pltpu.VMEM((_GCH, F), jnp.float32),
            pltpu.VMEM((_GCH, F), jnp.float32),
            pltpu.SemaphoreType.DMA,
        ],
    )
    def k(a_hbm, b_hbm, src_hbm, dst_hbm, out_hbm, si_v, di_v, ar_v, br_v, sem):
        wid = lax.axis_index("s") * _NC + lax.axis_index("c")
        base = wid * _EPT

        def chunk_body(ci, carry):
            off = base + ci * _GCH
            pltpu.sync_copy(src_hbm.at[pl.ds(off, _GCH)], si_v)
            pltpu.sync_copy(dst_hbm.at[pl.ds(off, _GCH)], di_v)
            pltpu.async_copy(a_hbm.at[si_v], ar_v, sem).wait()
            pltpu.async_copy(b_hbm.at[di_v], br_v, sem).wait()

            def row_body(i, c2):
                for j in range(F // 16):
                    sl = pl.ds(j * 16, 16)
                    ar_v[i, sl] = jnp.maximum(ar_v[i, sl] - br_v[i, sl], 0.0)
                return c2

            lax.fori_loop(0, _GCH, row_body, 0)
            pltpu.sync_copy(ar_v, out_hbm.at[pl.ds(off, _GCH)])
            return carry

        lax.fori_loop(0, _GNCH, chunk_body, 0)

    return k(A, B, src, dst)


def _node_tables_body(h_ref, pos_ref, wh_ref, wp_ref, b_ref, a_ref, b_out_ref):
    pq = jax.lax.dot_general(pos_ref[...], wp_ref[...], (((1,), (0,)), ((), ())),
                             preferred_element_type=jnp.float32)
    hq = jax.lax.dot_general(h_ref[...], wh_ref[...], (((1,), (0,)), ((), ())),
                             preferred_element_type=jnp.float32)
    a_ref[...] = hq + pq + b_ref[...]
    b_out_ref[...] = pq


def _node_tables(h, pos, Wh, Wp, b):
    n = h.shape[0]
    k = h.shape[1]
    grid = n // _NODE_BLK
    return pl.pallas_call(
        _node_tables_body,
        grid=(grid,),
        in_specs=[
            pl.BlockSpec((_NODE_BLK, k), lambda i: (i, 0)),
            pl.BlockSpec((_NODE_BLK, 3), lambda i: (i, 0)),
            pl.BlockSpec((k, F), lambda i: (0, 0)),
            pl.BlockSpec((3, F), lambda i: (0, 0)),
            pl.BlockSpec((1, F), lambda i: (0, 0)),
        ],
        out_specs=[
            pl.BlockSpec((_NODE_BLK, F), lambda i: (i, 0)),
            pl.BlockSpec((_NODE_BLK, F), lambda i: (i, 0)),
        ],
        out_shape=[
            jax.ShapeDtypeStruct((n, F), jnp.float32),
            jax.ShapeDtypeStruct((n, F), jnp.float32),
        ],
    )(h, pos, Wh, Wp, b[None, :])


def _edge_mlp_body(e_ref, wb_ref, bb_ref, out_ref):
    out_ref[...] = jax.lax.dot_general(
        e_ref[...], wb_ref[...], (((1,), (0,)), ((), ())),
        preferred_element_type=jnp.float32) + bb_ref[...]


def _edge_mlp(e, Wb, bb):
    ne = e.shape[0]
    grid = ne // _EDGE_BLK
    return pl.pallas_call(
        _edge_mlp_body,
        grid=(grid,),
        in_specs=[
            pl.BlockSpec((_EDGE_BLK, F), lambda i: (i, 0)),
            pl.BlockSpec((F, F), lambda i: (0, 0)),
            pl.BlockSpec((1, F), lambda i: (0, 0)),
        ],
        out_specs=pl.BlockSpec((_EDGE_BLK, F), lambda i: (i, 0)),
        out_shape=jax.ShapeDtypeStruct((ne, F), jnp.float32),
    )(e, Wb, bb[None, :])


def _layer(h, pos, src, dst, Wh, Wp, ba, Wb, bb):
    A, B = _node_tables(h, pos, Wh, Wp, ba)
    e = _gather_diff_relu(A, B, src, dst)
    msg = _edge_mlp(e, Wb, bb)
    agg = jax.ops.segment_max(msg, dst, num_segments=N_NODES)
    return jnp.maximum(agg, 0.0)


def _pool_classify_body(h_ref, wc_ref, bc_ref, out_ref):
    out_ref[...] = jax.lax.dot_general(
        h_ref[...], wc_ref[...], (((1,), (0,)), ((), ())),
        preferred_element_type=jnp.float32) + bc_ref[...]


def kernel(pos, edge_index, batch, W1a, b1a, W1b, b1b, W2a, b2a, W2b, b2b, Wc, bc):
    src = edge_index[0]
    dst = edge_index[1]
    h = _layer(pos, pos, src, dst, W1a[:3], W1a[3:], b1a, W1b, b1b)
    h = _layer(h, pos, src, dst, W2a[:F], W2a[F:], b2a, W2b, b2b)
    pooled = jnp.maximum(jax.ops.segment_max(h, batch, num_segments=64), 0.0)
    nc = Wc.shape[1]
    out = pl.pallas_call(
        _pool_classify_body,
        in_specs=[
            pl.BlockSpec((64, F), lambda: (0, 0)),
            pl.BlockSpec((F, nc), lambda: (0, 0)),
            pl.BlockSpec((1, nc), lambda: (0, 0)),
        ],
        out_specs=pl.BlockSpec((64, nc), lambda: (0, 0)),
        out_shape=jax.ShapeDtypeStruct((64, nc), jnp.float32),
    )(pooled, Wc, bc[None, :])
    return out
